# Initial kernel scaffold; baseline (speedup 1.0000x reference)
import jax
import jax.numpy as jnp
from jax.experimental import pallas as pl


def kernel(x, tables):
    raise NotImplementedError("write your pallas kernel here")


# speedup vs baseline: 1.6459x; 1.6459x over previous
import functools

import jax
import jax.numpy as jnp
from jax import lax
from jax.experimental import pallas as pl
from jax.experimental.pallas import tpu as pltpu
from jax.experimental.pallas import tpu_sc as plsc

NUM_FIELDS = 26
VOCAB = 100000
D_MODEL = 32
B, C, T = 1024, 1, 20
N = B * C * T * NUM_FIELDS
NW = 32
PER_W = N // NW
CHUNK = 1040
NCH = PER_W // CHUNK
L = 16
SL = CHUNK // L

_mesh = plsc.VectorSubcoreMesh(core_axis_name="c", subcore_axis_name="s")


@functools.partial(
    pl.kernel,
    mesh=_mesh,
    out_type=jax.ShapeDtypeStruct((N, D_MODEL), jnp.float32),
    scratch_types=[
        pltpu.VMEM((CHUNK,), jnp.int32),
        pltpu.VMEM((CHUNK,), jnp.int32),
        pltpu.VMEM((CHUNK, D_MODEL), jnp.float32),
        pltpu.SemaphoreType.DMA,
    ],
    compiler_params=pltpu.CompilerParams(use_tc_tiling_on_sc=False),
)
def _embed(x_hbm, tab_hbm, out_hbm, idx_v, offs_v, rows_v, sem):
    wid = lax.axis_index("s") * 2 + lax.axis_index("c")
    base = wid * PER_W

    def set_offs(j, carry):
        p = j * L + lax.iota(jnp.int32, L)
        offs_v[pl.ds(j * L, L)] = lax.rem(p, NUM_FIELDS) * VOCAB
        return carry

    lax.fori_loop(0, SL, set_offs, 0)

    def chunk_body(k, carry):
        cb = base + k * CHUNK
        pltpu.sync_copy(x_hbm.at[pl.ds(cb, CHUNK)], idx_v)

        def adj(j, c2):
            s = pl.ds(j * L, L)
            idx_v[s] = idx_v[s] + offs_v[s]
            return c2

        lax.fori_loop(0, SL, adj, 0)
        pltpu.async_copy(tab_hbm.at[idx_v], rows_v, sem).wait()
        pltpu.sync_copy(rows_v, out_hbm.at[pl.ds(cb, CHUNK)])
        return carry

    lax.fori_loop(0, NCH, chunk_body, 0)


def kernel(x, tables):
    xf = x.reshape(N).astype(jnp.int32)
    tf = tables.reshape(NUM_FIELDS * VOCAB, D_MODEL)
    out = _embed(xf, tf)
    return out.reshape(B, C, T, NUM_FIELDS, D_MODEL)


# speedup vs baseline: 1.6696x; 1.0144x over previous
import functools

import jax
import jax.numpy as jnp
from jax import lax
from jax.experimental import pallas as pl
from jax.experimental.pallas import tpu as pltpu
from jax.experimental.pallas import tpu_sc as plsc

NUM_FIELDS = 26
VOCAB = 100000
D_MODEL = 32
B, C, T = 1024, 1, 20
N = B * C * T * NUM_FIELDS
NW = 32
PER_W = N // NW
CHUNK = 416
NCH = PER_W // CHUNK
NBUF = 4
SUPER = NCH // NBUF
L = 16
SLICES = CHUNK // L

_mesh = plsc.VectorSubcoreMesh(core_axis_name="c", subcore_axis_name="s")


@functools.partial(
    pl.kernel,
    mesh=_mesh,
    out_type=jax.ShapeDtypeStruct((N, D_MODEL), jnp.float32),
    scratch_types=[
        pltpu.VMEM((PER_W,), jnp.int32),
        pltpu.VMEM((CHUNK, D_MODEL), jnp.float32),
        pltpu.VMEM((CHUNK, D_MODEL), jnp.float32),
        pltpu.VMEM((CHUNK, D_MODEL), jnp.float32),
        pltpu.VMEM((CHUNK, D_MODEL), jnp.float32),
        pltpu.SemaphoreType.DMA,
        pltpu.SemaphoreType.DMA,
        pltpu.SemaphoreType.DMA,
        pltpu.SemaphoreType.DMA,
        pltpu.SemaphoreType.DMA,
        pltpu.SemaphoreType.DMA,
        pltpu.SemaphoreType.DMA,
        pltpu.SemaphoreType.DMA,
    ],
    compiler_params=pltpu.CompilerParams(use_tc_tiling_on_sc=False),
)
def _embed(x_hbm, tab_hbm, out_hbm, idx_v, r0, r1, r2, r3,
           g0, g1, g2, g3, o0, o1, o2, o3):
    rows = (r0, r1, r2, r3)
    gsem = (g0, g1, g2, g3)
    osem = (o0, o1, o2, o3)
    wid = lax.axis_index("s") * 2 + lax.axis_index("c")
    base = wid * PER_W

    pltpu.sync_copy(x_hbm.at[pl.ds(base, PER_W)], idx_v)

    def adjust_chunk(j):
        def adj(s, carry):
            sl = j * SLICES + s
            span = pl.ds(sl * L, L)
            p = sl * L + lax.iota(jnp.int32, L)
            idx_v[span] = idx_v[span] + lax.rem(p, NUM_FIELDS) * VOCAB
            return carry
        lax.fori_loop(0, SLICES, adj, 0)

    def start_gather(j, b):
        pltpu.async_copy(
            tab_hbm.at[idx_v.at[pl.ds(j * CHUNK, CHUNK)]], rows[b], gsem[b])

    def wait_gather(j, b):
        pltpu.make_async_copy(
            tab_hbm.at[idx_v.at[pl.ds(j * CHUNK, CHUNK)]], rows[b], gsem[b]
        ).wait()

    def start_out(j, b):
        pltpu.async_copy(
            rows[b], out_hbm.at[pl.ds(base + j * CHUNK, CHUNK)], osem[b])

    def wait_out(j, b):
        pltpu.make_async_copy(
            rows[b], out_hbm.at[pl.ds(base + j * CHUNK, CHUNK)], osem[b]
        ).wait()

    def step(k, b, first, last):
        if not last:
            adjust_chunk(k + 2)
        wait_gather(k, b)
        start_out(k, b)
        if not last:
            bn = (b + 2) % NBUF
            if not first:
                wait_out(k - 2, bn)
            start_gather(k + 2, bn)

    adjust_chunk(0)
    adjust_chunk(1)
    start_gather(0, 0)
    start_gather(1, 1)

    step(0, 0, True, False)
    step(1, 1, True, False)
    step(2, 2, False, False)
    step(3, 3, False, False)

    def superstep(i, carry):
        k0 = i * NBUF
        for b in range(NBUF):
            step(k0 + b, b, False, False)
        return carry

    lax.fori_loop(1, SUPER - 1, superstep, 0)

    step((SUPER - 1) * NBUF + 0, 0, False, False)
    step((SUPER - 1) * NBUF + 1, 1, False, False)
    step((SUPER - 1) * NBUF + 2, 2, False, True)
    step((SUPER - 1) * NBUF + 3, 3, False, True)

    for b in range(NBUF):
        wait_out((SUPER - 1) * NBUF + b, b)


def kernel(x, tables):
    xf = x.reshape(N).astype(jnp.int32)
    tf = tables.reshape(NUM_FIELDS * VOCAB, D_MODEL)
    out = _embed(xf, tf)
    return out.reshape(B, C, T, NUM_FIELDS, D_MODEL)
